# baseline (speedup 1.0000x reference)
import jax
import jax.numpy as jnp
import numpy as np
from jax import lax
from jax.experimental import pallas as pl
from jax.experimental.pallas import tpu as pltpu
from jax.experimental.pallas import tpu_sc as plsc

B = 16384
S = 20
D = 64
NC = 2
NS = 16
NW = NC * NS
BPW = B // NW
HPW = BPW // 2
L = 16
CH = 32
NCHUNK = BPW // CH
IPG = 128
GPC = CH * S // IPG
NBUF = 2

MASK_HI = np.int32(-65536)


def _bf16_pair(words):
    ev = plsc.bitcast(lax.shift_left(words, 16), jnp.float32)
    od = plsc.bitcast(lax.bitwise_and(words, MASK_HI), jnp.float32)
    return ev, od


def _body(title2d, tok2d, title_table, token_table, out,
          k0, k1, t0, t1,
          tok_idx2, tok_idx, title_idx, out_buf, inv_v, nzf_v, row0_v,
          s0, s1):
    wid = lax.axis_index("s") * NC + lax.axis_index("c")
    base = wid * BPW
    krows = (k0, k1)
    trows = (t0, t1)
    sems = (s0, s1)

    pltpu.sync_copy(title2d.at[pl.ds(base, BPW)], title_idx)
    pltpu.sync_copy(token_table.at[0], row0_v)

    lanes = lax.iota(jnp.int32, L)

    for h2 in range(2):
        pltpu.sync_copy(tok2d.at[pl.ds(base + h2 * HPW, HPW), :], tok_idx2)

        def repack_body(w, carry):
            f0 = w * 128 + lanes
            for k in range(128 // L):
                f = f0 + k * L
                row = lax.shift_right_logical(f * np.int32(52429), 20)
                col = f - row * S
                tok_idx[h2 * (HPW * S // 128) + w, pl.ds(k * L, L)] = \
                    plsc.load_gather(tok_idx2, [row, col])
            return carry
        lax.fori_loop(0, HPW * S // 128, repack_body, 0)

        def count_body(g, carry):
            bvec = g * L + lanes
            cnt = jnp.zeros((L,), jnp.float32)
            for s in range(S):
                t = plsc.load_gather(
                    tok_idx2, [bvec, jnp.full((L,), s, jnp.int32)])
                cnt = cnt + (t != 0).astype(jnp.float32)
            inv_v[pl.ds(h2 * HPW + g * L, L)] = 1.0 / jnp.maximum(cnt, 1.0)
            nzf_v[pl.ds(h2 * HPW + g * L, L)] = float(S) - cnt
            return carry
        lax.fori_loop(0, HPW // L, count_body, 0)

    def fire(c, kb, tb, sm):
        for j in range(GPC):
            pltpu.async_copy(
                token_table.at[tok_idx.at[c * GPC + j]],
                kb.at[pl.ds(j * IPG, IPG)], sm)
        pltpu.async_copy(title_table.at[title_idx.at[pl.ds(c * CH, CH)]],
                         tb, sm)

    def drain(c, kb, tb, sm):
        for j in range(GPC):
            pltpu.make_async_copy(
                token_table.at[tok_idx.at[c * GPC + j]],
                kb.at[pl.ds(j * IPG, IPG)], sm).wait()
        pltpu.make_async_copy(title_table.at[title_idx.at[pl.ds(c * CH, CH)]],
                              tb, sm).wait()

    for c in range(NBUF - 1):
        fire(c, krows[c], trows[c], sems[c])

    row0 = []
    for h in range(2):
        w = plsc.bitcast(row0_v[pl.ds(h * 2 * L, 2 * L)], jnp.int32)
        row0 += list(_bf16_pair(w))

    def outer_body(c0, carry):
        for lane in range(NBUF):
            c = c0 * NBUF + lane
            kb, tb, sm = krows[lane], trows[lane], sems[lane]
            nxt = (lane + NBUF - 1) % NBUF

            @pl.when(c < NCHUNK - (NBUF - 1))
            def _():
                fire(c + NBUF - 1, krows[nxt], trows[nxt], sems[nxt])

            drain(c, kb, tb, sm)

            @plsc.parallel_loop(0, CH, unroll=2)
            def row_body(b):
                gb = c * CH + b
                rb = b * S
                for dv in range(D // L):
                    out_buf[gb, pl.ds(dv * L, L)] = tb[b, pl.ds(dv * L, L)]
                gidx = jnp.full((L,), gb, jnp.int32)
                ib = plsc.load_gather(inv_v, [gidx])
                nb = plsc.load_gather(nzf_v, [gidx])
                for h in range(2):
                    acc_e = jnp.zeros((L,), jnp.float32)
                    acc_o = jnp.zeros((L,), jnp.float32)
                    for s in range(S):
                        w = plsc.bitcast(
                            kb[rb + s, pl.ds(h * 2 * L, 2 * L)], jnp.int32)
                        ev, od = _bf16_pair(w)
                        acc_e = acc_e + ev
                        acc_o = acc_o + od
                    cols = D + h * 2 * L + lanes * 2
                    plsc.store_scatter(
                        out_buf, [gidx, cols],
                        (acc_e - nb * row0[2 * h]) * ib)
                    plsc.store_scatter(
                        out_buf, [gidx, cols + 1],
                        (acc_o - nb * row0[2 * h + 1]) * ib)
        return carry
    lax.fori_loop(0, NCHUNK // NBUF, outer_body, 0)

    pltpu.sync_copy(out_buf, out.at[pl.ds(base, BPW)])


def kernel(title_ids, token_ids, title_table, token_table):
    token_table = token_table.astype(jnp.bfloat16)
    mesh = plsc.VectorSubcoreMesh(core_axis_name="c", subcore_axis_name="s")
    f = pl.kernel(
        _body,
        out_type=jax.ShapeDtypeStruct((B, 2 * D), jnp.float32),
        mesh=mesh,
        compiler_params=pltpu.CompilerParams(
            needs_layout_passes=False, use_tc_tiling_on_sc=False),
        scratch_types=(
            [pltpu.VMEM((CH * S, D), jnp.bfloat16) for _ in range(NBUF)] +
            [pltpu.VMEM((CH, D), jnp.float32) for _ in range(NBUF)] +
            [
                pltpu.VMEM((HPW, S), jnp.int32),
                pltpu.VMEM((BPW * S // 128, 128), jnp.int32),
                pltpu.VMEM((BPW,), jnp.int32),
                pltpu.VMEM((BPW, 2 * D), jnp.float32),
                pltpu.VMEM((BPW,), jnp.float32),
                pltpu.VMEM((BPW,), jnp.float32),
                pltpu.VMEM((D,), jnp.bfloat16),
            ] +
            [pltpu.SemaphoreType.DMA for _ in range(NBUF)]
        ),
    )
    return f(title_ids, token_ids, title_table, token_table)


# speedup vs baseline: 1.1712x; 1.1712x over previous
import jax
import jax.numpy as jnp
import numpy as np
from jax import lax
from jax.experimental import pallas as pl
from jax.experimental.pallas import tpu as pltpu
from jax.experimental.pallas import tpu_sc as plsc

B = 16384
S = 20
D = 64
NC = 2
NS = 16
NW = NC * NS
BPW = B // NW
L = 16
CH = 16
NCHUNK = BPW // CH
IPG = 80
GPC = CH * S // IPG
NBUF = 2

MASK_HI = np.int32(-65536)
MASK_LO = np.int32(65535)


def _bf16_pair(words):
    ev = plsc.bitcast(lax.shift_left(words, 16), jnp.float32)
    od = plsc.bitcast(lax.bitwise_and(words, MASK_HI), jnp.float32)
    return ev, od


def _body(title2d, tok2d, title_table, token_table, out,
          k0, k1, t0, t1,
          tok_idx16, tok_idx, title_idx, out_buf, inv_v, nzf_v, row0_v,
          s0, s1):
    wid = lax.axis_index("s") * NC + lax.axis_index("c")
    base = wid * BPW
    krows = (k0, k1)
    trows = (t0, t1)
    sems = (s0, s1)

    pltpu.sync_copy(tok2d.at[pl.ds(wid * (BPW * S), BPW * S)], tok_idx16)
    pltpu.sync_copy(title2d.at[pl.ds(wid * BPW, BPW)], title_idx)
    pltpu.sync_copy(token_table.at[0], row0_v)

    lanes = lax.iota(jnp.int32, L)

    def widen_body(m, carry):
        w = plsc.bitcast(tok_idx16[pl.ds(m * 2 * L, 2 * L)], jnp.int32)
        pos = m * 2 * L + lanes * 2
        plsc.store_scatter(tok_idx, [pos], lax.bitwise_and(w, MASK_LO))
        plsc.store_scatter(tok_idx, [pos + 1],
                           lax.shift_right_logical(w, 16))
        return carry
    lax.fori_loop(0, BPW * S // (2 * L), widen_body, 0)

    def fire(c, kb, tb, sm):
        for j in range(GPC):
            pltpu.async_copy(
                token_table.at[tok_idx.at[pl.ds(c * (CH * S) + j * IPG, IPG)]],
                kb.at[pl.ds(j * IPG, IPG)], sm)
        pltpu.async_copy(title_table.at[title_idx.at[pl.ds(c * CH, CH)]],
                         tb, sm)

    def drain(c, kb, tb, sm):
        for j in range(GPC):
            pltpu.make_async_copy(
                token_table.at[tok_idx.at[pl.ds(c * (CH * S) + j * IPG, IPG)]],
                kb.at[pl.ds(j * IPG, IPG)], sm).wait()
        pltpu.make_async_copy(title_table.at[title_idx.at[pl.ds(c * CH, CH)]],
                              tb, sm).wait()

    for c in range(NBUF - 1):
        fire(c, krows[c], trows[c], sems[c])

    def count_body(g, carry):
        flat0 = g * (L * S) + lanes * S
        cnt = jnp.zeros((L,), jnp.float32)
        for s in range(S):
            t = plsc.load_gather(tok_idx, [flat0 + s])
            cnt = cnt + (t != 0).astype(jnp.float32)
        inv_v[pl.ds(g * L, L)] = 1.0 / jnp.maximum(cnt, 1.0)
        nzf_v[pl.ds(g * L, L)] = float(S) - cnt
        return carry
    lax.fori_loop(0, BPW // L, count_body, 0)

    row0 = []
    for h in range(2):
        w = plsc.bitcast(row0_v[pl.ds(h * 2 * L, 2 * L)], jnp.int32)
        row0 += list(_bf16_pair(w))

    def outer_body(c0, carry):
        for lane in range(NBUF):
            c = c0 * NBUF + lane
            kb, tb, sm = krows[lane], trows[lane], sems[lane]
            nxt = (lane + NBUF - 1) % NBUF

            @pl.when(c < NCHUNK - (NBUF - 1))
            def _():
                fire(c + NBUF - 1, krows[nxt], trows[nxt], sems[nxt])

            drain(c, kb, tb, sm)

            @plsc.parallel_loop(0, CH, unroll=2)
            def row_body(b):
                gb = c * CH + b
                rb = b * S
                for dv in range(D // L):
                    out_buf[gb, pl.ds(dv * L, L)] = tb[b, pl.ds(dv * L, L)]
                gidx = jnp.full((L,), gb, jnp.int32)
                ib = plsc.load_gather(inv_v, [gidx])
                nb = plsc.load_gather(nzf_v, [gidx])
                for h in range(2):
                    acc_e = jnp.zeros((L,), jnp.float32)
                    acc_o = jnp.zeros((L,), jnp.float32)
                    for s in range(S):
                        w = plsc.bitcast(
                            kb[rb + s, pl.ds(h * 2 * L, 2 * L)], jnp.int32)
                        ev, od = _bf16_pair(w)
                        acc_e = acc_e + ev
                        acc_o = acc_o + od
                    cols = D + h * 2 * L + lanes * 2
                    plsc.store_scatter(
                        out_buf, [gidx, cols],
                        (acc_e - nb * row0[2 * h]) * ib)
                    plsc.store_scatter(
                        out_buf, [gidx, cols + 1],
                        (acc_o - nb * row0[2 * h + 1]) * ib)
        return carry
    lax.fori_loop(0, NCHUNK // NBUF, outer_body, 0)

    pltpu.sync_copy(out_buf, out.at[pl.ds(base, BPW)])


def kernel(title_ids, token_ids, title_table, token_table):
    tok2d = token_ids.astype(jnp.int16).reshape(B * S)
    title2d = title_ids
    token_table = token_table.astype(jnp.bfloat16)
    mesh = plsc.VectorSubcoreMesh(core_axis_name="c", subcore_axis_name="s")
    f = pl.kernel(
        _body,
        out_type=jax.ShapeDtypeStruct((B, 2 * D), jnp.float32),
        mesh=mesh,
        compiler_params=pltpu.CompilerParams(
            needs_layout_passes=False, use_tc_tiling_on_sc=False),
        scratch_types=(
            [pltpu.VMEM((CH * S, D), jnp.bfloat16) for _ in range(NBUF)] +
            [pltpu.VMEM((CH, D), jnp.float32) for _ in range(NBUF)] +
            [
                pltpu.VMEM((BPW * S,), jnp.int16),
                pltpu.VMEM((BPW * S,), jnp.int32),
                pltpu.VMEM((BPW,), jnp.int32),
                pltpu.VMEM((BPW, 2 * D), jnp.float32),
                pltpu.VMEM((BPW,), jnp.float32),
                pltpu.VMEM((BPW,), jnp.float32),
                pltpu.VMEM((D,), jnp.bfloat16),
            ] +
            [pltpu.SemaphoreType.DMA for _ in range(NBUF)]
        ),
    )
    return f(title2d, tok2d, title_table, token_table)
